# baseline (speedup 1.0000x reference)
import functools

import jax
import jax.numpy as jnp
from jax import lax
from jax.experimental import pallas as pl
from jax.experimental.pallas import tpu as pltpu
from jax.experimental.pallas import tpu_sc as plsc

LOG_ZERO = -10000000.0
ROWS = 64
COLS = 100000
K = 16
ROW_BLK = 8
COL_BLK = 12500

NC, NS, L = 2, 16, 16
NW = NC * NS
ROWS_PER_W = ROWS // NW
NCHUNK = COLS // L
NEG = -3.4e38


def _sc_topk_body(scores_hbm, topv_hbm, topi_hbm, row_v, tv_v, ti_v):
    wid = lax.axis_index("s") * NC + lax.axis_index("c")
    iota = lax.iota(jnp.int32, L)

    for rr in range(ROWS_PER_W):
        r = wid * ROWS_PER_W + rr
        pltpu.sync_copy(scores_hbm.at[r], row_v)

        zeros = jnp.zeros((L,), jnp.int32)

        def step(c, carry):
            bv, bi, tv = carry
            x = row_v[pl.ds(c * L, L)]
            hit = jnp.any(x > tv)

            def merge(args):
                bv, bi, _ = args
                xs, xi = plsc.sort_key_val(x, iota + c * L)
                xr = lax.rev(xs, (0,))
                xir = lax.rev(xi, (0,))
                keep = bv >= xr
                nv = jnp.where(keep, bv, xr)
                ni = jnp.where(keep, bi, xir)
                nv, ni = plsc.sort_key_val(nv, ni)
                return nv, ni, nv[zeros]

            return lax.cond(hit, merge, lambda a: a, (bv, bi, tv))

        bv0 = jnp.full((L,), NEG, jnp.float32)
        bi0 = jnp.zeros((L,), jnp.int32)
        bv, bi, _ = lax.fori_loop(0, NCHUNK, step, (bv0, bi0, bv0))

        tv_v[...] = lax.rev(bv, (0,))
        ti_v[...] = lax.rev(bi, (0,))
        pltpu.sync_copy(tv_v, topv_hbm.at[r])
        pltpu.sync_copy(ti_v, topi_hbm.at[r])


def _sc_topk(scores):
    return pl.kernel(
        _sc_topk_body,
        out_type=[
            jax.ShapeDtypeStruct((ROWS, K), jnp.float32),
            jax.ShapeDtypeStruct((ROWS, K), jnp.int32),
        ],
        mesh=plsc.VectorSubcoreMesh(core_axis_name="c", subcore_axis_name="s"),
        scratch_types=[
            pltpu.VMEM((COLS,), jnp.float32),
            pltpu.VMEM((K,), jnp.float32),
            pltpu.VMEM((K,), jnp.int32),
        ],
        compiler_params=pltpu.CompilerParams(needs_layout_passes=False),
    )(scores)


def _tc_lse_body(x_ref, topv_raw_ref, lse_ref, topv_ref):
    x = x_ref[...]
    m = jnp.max(x, axis=-1, keepdims=True)
    s = jnp.sum(jnp.exp(x - m), axis=-1, keepdims=True)
    lse = m + jnp.log(s)
    lse_ref[...] = lse
    topv_ref[...] = topv_raw_ref[...] - lse


def _tc_lse(scores, topv_raw):
    return pl.pallas_call(
        _tc_lse_body,
        grid=(ROWS // ROW_BLK,),
        in_specs=[
            pl.BlockSpec((ROW_BLK, COLS), lambda i: (i, 0)),
            pl.BlockSpec((ROW_BLK, K), lambda i: (i, 0)),
        ],
        out_specs=[
            pl.BlockSpec((ROW_BLK, 1), lambda i: (i, 0)),
            pl.BlockSpec((ROW_BLK, K), lambda i: (i, 0)),
        ],
        out_shape=[
            jax.ShapeDtypeStruct((ROWS, 1), jnp.float32),
            jax.ShapeDtypeStruct((ROWS, K), jnp.float32),
        ],
    )(scores, topv_raw)


def _tc_mask_body(x_ref, lse_ref, topv_raw_ref, out_ref):
    lse = lse_ref[...]
    thresh = topv_raw_ref[:, K - 1:K] - lse
    logp = x_ref[...] - lse
    out_ref[...] = jnp.where(logp >= thresh, logp, LOG_ZERO)


def _tc_mask(scores, lse, topv_raw):
    return pl.pallas_call(
        _tc_mask_body,
        grid=(ROWS // ROW_BLK,),
        in_specs=[
            pl.BlockSpec((ROW_BLK, COLS), lambda i: (i, 0)),
            pl.BlockSpec((ROW_BLK, 1), lambda i: (i, 0)),
            pl.BlockSpec((ROW_BLK, K), lambda i: (i, 0)),
        ],
        out_specs=pl.BlockSpec((ROW_BLK, COLS), lambda i: (i, 0)),
        out_shape=jax.ShapeDtypeStruct((ROWS, COLS), jnp.float32),
    )(scores, lse, topv_raw)


def kernel(scores, k):
    topv_raw, topi_raw = _sc_topk(scores)
    lse, topv = _tc_lse(scores, topv_raw)
    masked = _tc_mask(scores, lse, topv_raw)
    topi = topi_raw + jnp.asarray(k - K, dtype=jnp.int32)
    return masked, topv, topi


# speedup vs baseline: 2.2095x; 2.2095x over previous
import functools

import jax
import jax.numpy as jnp
from jax import lax
from jax.experimental import pallas as pl
from jax.experimental.pallas import tpu as pltpu
from jax.experimental.pallas import tpu_sc as plsc

LOG_ZERO = -10000000.0
ROWS = 64
COLS = 100000
K = 16
ROW_BLK = 8
COL_BLK = 12500

NC, NS, L = 2, 16, 16
NW = NC * NS
ROWS_PER_W = ROWS // NW
NCHUNK = COLS // L
NEG = -3.4e38


GRP = 10
U2 = 8
CAP = 1024


def _sc_topk_body(scores_hbm, topv_hbm, topi_hbm, row_v, ci_v, cw_v, tv_v, ti_v):
    wid = lax.axis_index("s") * NC + lax.axis_index("c")
    iota = lax.iota(jnp.int32, L)
    zeros = jnp.zeros((L,), jnp.int32)
    fifteen = jnp.full((L,), L - 1, jnp.int32)

    for rr in range(ROWS_PER_W):
        r = wid * ROWS_PER_W + rr
        pltpu.sync_copy(scores_hbm.at[r], row_v)

        def pa(g, acc):
            base = g * (GRP * L)
            gm = row_v[pl.ds(base, L)]
            for u in range(1, GRP):
                gm = jnp.maximum(gm, row_v[pl.ds(base + u * L, L)])
            return jnp.maximum(acc, gm)

        acc = lax.fori_loop(0, NCHUNK // GRP, pa,
                            jnp.full((L,), NEG, jnp.float32))
        tlb = jnp.full((L,), jnp.min(acc), jnp.float32)

        def pb(g, ov):
            base = g * (U2 * L)
            for u in range(U2):
                cbase = base + u * L
                x = row_v[pl.ds(cbase, L)]
                m = x >= tlb
                pc = plsc.cumsum(jnp.where(m, 1, 0))
                pos = ov + pc - 1
                msk = m & (pos < CAP)
                plsc.store_scatter(ci_v, [pos], iota + cbase, mask=msk)
                ov = ov + pc[fifteen]
            return ov

        ov = lax.fori_loop(0, NCHUNK // U2, pb, zeros)
        cnt = jnp.minimum(ov[0], CAP)
        nch = (cnt + L - 1) // L

        def pg(w, _):
            valid = (w * L + iota) < cnt
            ci = jnp.where(valid, ci_v[pl.ds(w * L, L)], 0)
            ci_v[pl.ds(w * L, L)] = ci
            cv = plsc.load_gather(row_v, [ci])
            cw_v[pl.ds(w * L, L)] = jnp.where(valid, cv, NEG)
            return 0

        lax.fori_loop(0, nch, pg, 0)

        negs = jnp.full((L,), NEG, jnp.float32)

        def sel(j, carry):
            tva, tia = carry

            def scan(w, c2):
                bv, bi = c2
                cv = cw_v[pl.ds(w * L, L)]
                ci = ci_v[pl.ds(w * L, L)]
                better = (cv > bv) | ((cv == bv) & (ci < bi))
                return (jnp.where(better, cv, bv),
                        jnp.where(better, ci, bi))

            bv, bi = lax.fori_loop(0, nch, scan, (negs, zeros))
            for s in (8, 4, 2, 1):
                perm = jnp.bitwise_xor(iota, s)
                ov2, oi2 = bv[perm], bi[perm]
                better = (ov2 > bv) | ((ov2 == bv) & (oi2 < bi))
                bv = jnp.where(better, ov2, bv)
                bi = jnp.where(better, oi2, bi)
            tva = jnp.where(iota == j, bv, tva)
            tia = jnp.where(iota == j, bi, tia)

            def evict(w, _):
                cv = cw_v[pl.ds(w * L, L)]
                ci = ci_v[pl.ds(w * L, L)]
                cw_v[pl.ds(w * L, L)] = jnp.where(ci == bi, NEG, cv)
                return 0

            lax.fori_loop(0, nch, evict, 0)
            return tva, tia

        tva, tia = lax.fori_loop(0, K, sel, (negs, zeros))

        tv_v[...] = tva
        ti_v[...] = tia
        pltpu.sync_copy(tv_v, topv_hbm.at[r])
        pltpu.sync_copy(ti_v, topi_hbm.at[r])


def _sc_topk(scores):
    return pl.kernel(
        _sc_topk_body,
        out_type=[
            jax.ShapeDtypeStruct((ROWS, K), jnp.float32),
            jax.ShapeDtypeStruct((ROWS, K), jnp.int32),
        ],
        mesh=plsc.VectorSubcoreMesh(core_axis_name="c", subcore_axis_name="s"),
        scratch_types=[
            pltpu.VMEM((COLS,), jnp.float32),
            pltpu.VMEM((CAP,), jnp.int32),
            pltpu.VMEM((CAP,), jnp.float32),
            pltpu.VMEM((K,), jnp.float32),
            pltpu.VMEM((K,), jnp.int32),
        ],
        compiler_params=pltpu.CompilerParams(needs_layout_passes=False),
    )(scores)


def _tc_lse_body(x_ref, topv_raw_ref, lse_ref, topv_ref):
    x = x_ref[...]
    m = jnp.max(x, axis=-1, keepdims=True)
    s = jnp.sum(jnp.exp(x - m), axis=-1, keepdims=True)
    lse = m + jnp.log(s)
    lse_ref[...] = lse
    topv_ref[...] = topv_raw_ref[...] - lse


def _tc_lse(scores, topv_raw):
    return pl.pallas_call(
        _tc_lse_body,
        grid=(ROWS // ROW_BLK,),
        in_specs=[
            pl.BlockSpec((ROW_BLK, COLS), lambda i: (i, 0)),
            pl.BlockSpec((ROW_BLK, K), lambda i: (i, 0)),
        ],
        out_specs=[
            pl.BlockSpec((ROW_BLK, 1), lambda i: (i, 0)),
            pl.BlockSpec((ROW_BLK, K), lambda i: (i, 0)),
        ],
        out_shape=[
            jax.ShapeDtypeStruct((ROWS, 1), jnp.float32),
            jax.ShapeDtypeStruct((ROWS, K), jnp.float32),
        ],
    )(scores, topv_raw)


def _tc_mask_body(x_ref, lse_ref, topv_raw_ref, out_ref):
    lse = lse_ref[...]
    thresh = topv_raw_ref[:, K - 1:K] - lse
    logp = x_ref[...] - lse
    out_ref[...] = jnp.where(logp >= thresh, logp, LOG_ZERO)


def _tc_mask(scores, lse, topv_raw):
    return pl.pallas_call(
        _tc_mask_body,
        grid=(ROWS // ROW_BLK,),
        in_specs=[
            pl.BlockSpec((ROW_BLK, COLS), lambda i: (i, 0)),
            pl.BlockSpec((ROW_BLK, 1), lambda i: (i, 0)),
            pl.BlockSpec((ROW_BLK, K), lambda i: (i, 0)),
        ],
        out_specs=pl.BlockSpec((ROW_BLK, COLS), lambda i: (i, 0)),
        out_shape=jax.ShapeDtypeStruct((ROWS, COLS), jnp.float32),
    )(scores, lse, topv_raw)


def kernel(scores, k):
    topv_raw, topi_raw = _sc_topk(scores)
    lse, topv = _tc_lse(scores, topv_raw)
    masked = _tc_mask(scores, lse, topv_raw)
    topi = topi_raw + jnp.asarray(k - K, dtype=jnp.int32)
    return masked, topv, topi


# speedup vs baseline: 5.3293x; 2.4120x over previous
import functools

import jax
import jax.numpy as jnp
from jax import lax
from jax.experimental import pallas as pl
from jax.experimental.pallas import tpu as pltpu
from jax.experimental.pallas import tpu_sc as plsc

LOG_ZERO = -10000000.0
ROWS = 64
COLS = 100000
K = 16
ROW_BLK = 8
COL_BLK = 12500

NC, NS, L = 2, 16, 16
NW = NC * NS
ROWS_PER_W = ROWS // NW
NCHUNK = COLS // L
NEG = -3.4e38


GRP = 10
NGRP = NCHUNK // GRP
U2 = 5
CAP = 1024


def _sc_topk_body(scores_hbm, topv_hbm, topi_hbm,
                  row_v, gmax_v, co_v, ci_v, cw_v, tv_v, ti_v):
    wid = lax.axis_index("s") * NC + lax.axis_index("c")
    iota = lax.iota(jnp.int32, L)
    zeros = jnp.zeros((L,), jnp.int32)
    fifteen = jnp.full((L,), L - 1, jnp.int32)

    for rr in range(ROWS_PER_W):
        r = wid * ROWS_PER_W + rr
        pltpu.sync_copy(scores_hbm.at[r], row_v)

        def pa(g, acc):
            base = g * (GRP * L)
            gm = row_v[pl.ds(base, L)]
            for u in range(1, GRP):
                gm = jnp.maximum(gm, row_v[pl.ds(base + u * L, L)])
            gmax_v[pl.ds(g * L, L)] = gm
            return jnp.maximum(acc, gm)

        acc = lax.fori_loop(0, NGRP, pa, jnp.full((L,), NEG, jnp.float32))
        tlb = jnp.full((L,), jnp.min(acc), jnp.float32)

        def pb1(g, ov):
            base = g * (U2 * L)
            for u in range(U2):
                cbase = base + u * L
                x = gmax_v[pl.ds(cbase, L)]
                m = x >= tlb
                pc = plsc.cumsum(jnp.where(m, 1, 0))
                pos = ov + pc - 1
                msk = m & (pos < CAP)
                plsc.store_scatter(co_v, [pos], iota + cbase, mask=msk)
                ov = ov + pc[fifteen]
            return ov

        hv = lax.fori_loop(0, NGRP // U2, pb1, zeros)
        hcnt = jnp.minimum(hv[0], CAP)
        nhch = (hcnt + L - 1) // L

        def pb2(h, ov):
            valid = (h * L + iota) < hcnt
            co = jnp.where(valid, co_v[pl.ds(h * L, L)], 0)
            cg = co >> 4
            cl = co & (L - 1)
            base = cg * (GRP * L) + cl
            for u in range(GRP):
                gi = base + u * L
                vals = plsc.load_gather(row_v, [gi])
                m = (vals >= tlb) & valid
                pc = plsc.cumsum(jnp.where(m, 1, 0))
                pos = ov + pc - 1
                msk = m & (pos < CAP)
                plsc.store_scatter(ci_v, [pos], gi, mask=msk)
                ov = ov + pc[fifteen]
            return ov

        ov = lax.fori_loop(0, nhch, pb2, zeros)
        cnt = jnp.minimum(ov[0], CAP)
        nch = (cnt + L - 1) // L

        def pg(w, _):
            valid = (w * L + iota) < cnt
            ci = jnp.where(valid, ci_v[pl.ds(w * L, L)], 0)
            ci_v[pl.ds(w * L, L)] = ci
            cv = plsc.load_gather(row_v, [ci])
            cw_v[pl.ds(w * L, L)] = jnp.where(valid, cv, NEG)
            return 0

        lax.fori_loop(0, nch, pg, 0)

        negs = jnp.full((L,), NEG, jnp.float32)

        def sel(j, carry):
            tva, tia = carry

            def scan(w, c2):
                bv, bi = c2
                cv = cw_v[pl.ds(w * L, L)]
                ci = ci_v[pl.ds(w * L, L)]
                better = (cv > bv) | ((cv == bv) & (ci < bi))
                return (jnp.where(better, cv, bv),
                        jnp.where(better, ci, bi))

            bv, bi = lax.fori_loop(0, nch, scan, (negs, zeros))
            for s in (8, 4, 2, 1):
                perm = jnp.bitwise_xor(iota, s)
                ov2, oi2 = bv[perm], bi[perm]
                better = (ov2 > bv) | ((ov2 == bv) & (oi2 < bi))
                bv = jnp.where(better, ov2, bv)
                bi = jnp.where(better, oi2, bi)
            tva = jnp.where(iota == j, bv, tva)
            tia = jnp.where(iota == j, bi, tia)

            def evict(w, _):
                cv = cw_v[pl.ds(w * L, L)]
                ci = ci_v[pl.ds(w * L, L)]
                cw_v[pl.ds(w * L, L)] = jnp.where(ci == bi, NEG, cv)
                return 0

            lax.fori_loop(0, nch, evict, 0)
            return tva, tia

        tva, tia = lax.fori_loop(0, K, sel, (negs, zeros))

        tv_v[...] = tva
        ti_v[...] = tia
        pltpu.sync_copy(tv_v, topv_hbm.at[r])
        pltpu.sync_copy(ti_v, topi_hbm.at[r])


def _sc_topk(scores):
    return pl.kernel(
        _sc_topk_body,
        out_type=[
            jax.ShapeDtypeStruct((ROWS, K), jnp.float32),
            jax.ShapeDtypeStruct((ROWS, K), jnp.int32),
        ],
        mesh=plsc.VectorSubcoreMesh(core_axis_name="c", subcore_axis_name="s"),
        scratch_types=[
            pltpu.VMEM((COLS,), jnp.float32),
            pltpu.VMEM((NGRP * L,), jnp.float32),
            pltpu.VMEM((CAP,), jnp.int32),
            pltpu.VMEM((CAP,), jnp.int32),
            pltpu.VMEM((CAP,), jnp.float32),
            pltpu.VMEM((K,), jnp.float32),
            pltpu.VMEM((K,), jnp.int32),
        ],
        compiler_params=pltpu.CompilerParams(needs_layout_passes=False),
    )(scores)


def _tc_lse_body(x_ref, topv_raw_ref, lse_ref, topv_ref):
    x = x_ref[...]
    m = jnp.max(x, axis=-1, keepdims=True)
    s = jnp.sum(jnp.exp(x - m), axis=-1, keepdims=True)
    lse = m + jnp.log(s)
    lse_ref[...] = lse
    topv_ref[...] = topv_raw_ref[...] - lse


def _tc_lse(scores, topv_raw):
    return pl.pallas_call(
        _tc_lse_body,
        grid=(ROWS // ROW_BLK,),
        in_specs=[
            pl.BlockSpec((ROW_BLK, COLS), lambda i: (i, 0)),
            pl.BlockSpec((ROW_BLK, K), lambda i: (i, 0)),
        ],
        out_specs=[
            pl.BlockSpec((ROW_BLK, 1), lambda i: (i, 0)),
            pl.BlockSpec((ROW_BLK, K), lambda i: (i, 0)),
        ],
        out_shape=[
            jax.ShapeDtypeStruct((ROWS, 1), jnp.float32),
            jax.ShapeDtypeStruct((ROWS, K), jnp.float32),
        ],
    )(scores, topv_raw)


def _tc_mask_body(x_ref, lse_ref, topv_raw_ref, out_ref):
    lse = lse_ref[...]
    thresh = topv_raw_ref[:, K - 1:K] - lse
    logp = x_ref[...] - lse
    out_ref[...] = jnp.where(logp >= thresh, logp, LOG_ZERO)


def _tc_mask(scores, lse, topv_raw):
    return pl.pallas_call(
        _tc_mask_body,
        grid=(ROWS // ROW_BLK,),
        in_specs=[
            pl.BlockSpec((ROW_BLK, COLS), lambda i: (i, 0)),
            pl.BlockSpec((ROW_BLK, 1), lambda i: (i, 0)),
            pl.BlockSpec((ROW_BLK, K), lambda i: (i, 0)),
        ],
        out_specs=pl.BlockSpec((ROW_BLK, COLS), lambda i: (i, 0)),
        out_shape=jax.ShapeDtypeStruct((ROWS, COLS), jnp.float32),
    )(scores, lse, topv_raw)


def kernel(scores, k):
    topv_raw, topi_raw = _sc_topk(scores)
    lse, topv = _tc_lse(scores, topv_raw)
    masked = _tc_mask(scores, lse, topv_raw)
    topi = topi_raw + jnp.asarray(k - K, dtype=jnp.int32)
    return masked, topv, topi
